# baseline (speedup 1.0000x reference)
import functools

import jax
import jax.numpy as jnp
from jax import lax
from jax.experimental import pallas as pl
from jax.experimental.pallas import tpu as pltpu
from jax.experimental.pallas import tpu_sc as plsc

N = 10000
E = 320000
D = 128
B = 16
L = 2048

NC = 2
NS = 16
NW = NC * NS
CH = 128
NCHUNK = 80
EPW = NCHUNK * CH
EPAD = NW * EPW
NPAD = 10240
PADN = NPAD - 8
RPT = NPAD // NS

TQ = B * L
QP = TQ // NW
QCH = 128
QNCH = QP // QCH

_mesh = functools.partial(
    plsc.VectorSubcoreMesh,
    core_axis_name="c", subcore_axis_name="s", num_cores=NC, num_subcores=NS)


ECORE = EPAD // NC
EPT = ECORE // NS


@functools.partial(
    pl.kernel,
    out_type=jax.ShapeDtypeStruct((NC, NPAD, D), jnp.float32),
    mesh=_mesh(),
    compiler_params=pltpu.CompilerParams(needs_layout_passes=False),
    scratch_types=[
        pltpu.VMEM((EPT,), jnp.int32),
        pltpu.VMEM((NPAD,), jnp.float32),
        pltpu.VMEM((RPT,), jnp.float32),
        pltpu.VMEM((RPT,), jnp.float32),
        pltpu.VMEM((RPT, D), jnp.float32),
        pltpu.VMEM_SHARED((NS, NPAD), jnp.float32),
    ],
)
def _deg_kernel(dstf_hbm, deg_out, dst_vm, acc, rsum, tmp, outbuf, part_sp):
    c = lax.axis_index("c")
    s = lax.axis_index("s")
    pltpu.sync_copy(dstf_hbm.at[c, pl.ds(s * EPT, EPT)], dst_vm)

    zeros16 = jnp.zeros((16,), jnp.float32)

    def zero(i, carry):
        acc[pl.ds(i * 16, 16)] = zeros16
        return carry

    lax.fori_loop(0, NPAD // 16, zero, 0)

    ones16 = jnp.ones((16,), jnp.float32)

    def hist(v, carry):
        idx = dst_vm[pl.ds(v * 16, 16)]
        plsc.addupdate_scatter(acc, [idx], ones16)
        return carry

    lax.fori_loop(0, EPT // 16, hist, 0)

    pltpu.sync_copy(acc, part_sp.at[s])
    plsc.subcore_barrier()

    base = s * RPT

    def red0(i, carry):
        rsum[pl.ds(i * 16, 16)] = zeros16
        return carry

    lax.fori_loop(0, RPT // 16, red0, 0)

    for r in range(NS):
        pltpu.sync_copy(part_sp.at[r, pl.ds(base, RPT)], tmp)

        def radd(i, carry):
            rsum[pl.ds(i * 16, 16)] = (rsum[pl.ds(i * 16, 16)]
                                       + tmp[pl.ds(i * 16, 16)])
            return carry

        lax.fori_loop(0, RPT // 16, radd, 0)

    def bcast(g, carry):
        vec = rsum[pl.ds(g * 16, 16)]
        for lane in range(16):
            splat = jnp.full((16,), vec[lane], jnp.float32)
            for k in range(D // 16):
                outbuf[g * 16 + lane, pl.ds(k * 16, 16)] = splat
        return carry

    lax.fori_loop(0, RPT // 16, bcast, 0)
    pltpu.sync_copy(outbuf, deg_out.at[c, pl.ds(base, RPT)])


@functools.partial(
    pl.kernel,
    out_type=jax.ShapeDtypeStruct((NC, NPAD, D), jnp.float32),
    mesh=_mesh(),
    scratch_types=[
        pltpu.VMEM((NCHUNK // 2, CH), jnp.int32),
        pltpu.VMEM((NCHUNK // 2, CH), jnp.int32),
        pltpu.VMEM((CH, D), jnp.float32),
        pltpu.VMEM((CH, D), jnp.float32),
        pltpu.SemaphoreType.DMA,
        pltpu.SemaphoreType.DMA,
        pltpu.SemaphoreType.DMA,
        pltpu.SemaphoreType.DMA,
        pltpu.VMEM_SHARED((NPAD, D), jnp.float32),
    ],
)
def _msg_kernel(y_hbm, src_hbm, dst_hbm, zeros_hbm, s_out, src_vm, dst_vm,
                buf_a, buf_b, gsem_a, gsem_b, ssem_a, ssem_b, s_sp):
    c = lax.axis_index("c")
    s = lax.axis_index("s")
    base = s * RPT
    pltpu.sync_copy(zeros_hbm, s_sp.at[pl.ds(base, RPT)])
    plsc.subcore_barrier()

    bufs = (buf_a, buf_b)
    gsems = (gsem_a, gsem_b)
    ssems = (ssem_a, ssem_b)
    half = NCHUNK // 2

    for p in range(2):
        pltpu.sync_copy(src_hbm.at[c, s, pl.ds(p * half, half)], src_vm)
        pltpu.sync_copy(dst_hbm.at[c, s, pl.ds(p * half, half)], dst_vm)
        pltpu.async_copy(y_hbm.at[src_vm.at[0]], buf_a, gsem_a)
        pltpu.async_copy(y_hbm.at[src_vm.at[1]], buf_b, gsem_b)

        def body(g, carry):
            for k in range(2):
                j = g * 2 + k
                buf, gsem, ssem = bufs[k], gsems[k], ssems[k]
                pltpu.make_async_copy(y_hbm.at[src_vm.at[j]], buf,
                                      gsem).wait()
                pltpu.async_copy(buf, s_sp.at[dst_vm.at[j]], ssem, add=True)

                @pl.when(j + 2 < half)
                def _():
                    pltpu.make_async_copy(buf, s_sp.at[dst_vm.at[j]],
                                          ssem).wait()
                    pltpu.async_copy(y_hbm.at[src_vm.at[j + 2]], buf, gsem)
            return carry

        lax.fori_loop(0, half // 2, body, 0)
        pltpu.make_async_copy(buf_a, s_sp.at[dst_vm.at[half - 2]],
                              ssem_a).wait()
        pltpu.make_async_copy(buf_b, s_sp.at[dst_vm.at[half - 1]],
                              ssem_b).wait()

    plsc.subcore_barrier()
    pltpu.sync_copy(s_sp.at[pl.ds(base, RPT)], s_out.at[c, pl.ds(base, RPT)])


@functools.partial(
    pl.kernel,
    out_type=jax.ShapeDtypeStruct((TQ, D), jnp.float32),
    mesh=_mesh(),
    scratch_types=[
        pltpu.VMEM((QNCH, QCH), jnp.int32),
        pltpu.VMEM((QCH, D), jnp.float32),
        pltpu.VMEM((QCH, D), jnp.float32),
        pltpu.SemaphoreType.DMA,
        pltpu.SemaphoreType.DMA,
        pltpu.SemaphoreType.DMA,
        pltpu.SemaphoreType.DMA,
    ],
)
def _traj_kernel(road_hbm, idx_hbm, out_hbm, idx_vm, buf_a, buf_b,
                 gsem_a, gsem_b, osem_a, osem_b):
    c = lax.axis_index("c")
    s = lax.axis_index("s")
    wid = s * NC + c
    pltpu.sync_copy(idx_hbm.at[wid], idx_vm)
    obase = wid * QP

    bufs = (buf_a, buf_b)
    gsems = (gsem_a, gsem_b)
    osems = (osem_a, osem_b)
    pltpu.async_copy(road_hbm.at[idx_vm.at[0]], buf_a, gsem_a)
    pltpu.async_copy(road_hbm.at[idx_vm.at[1]], buf_b, gsem_b)

    def body(g, carry):
        for k in range(2):
            j = g * 2 + k
            buf, gsem, osem = bufs[k], gsems[k], osems[k]
            dst = out_hbm.at[pl.ds(obase + j * QCH, QCH)]
            pltpu.make_async_copy(road_hbm.at[idx_vm.at[j]], buf, gsem).wait()
            pltpu.async_copy(buf, dst, osem)

            @pl.when(j + 2 < QNCH)
            def _():
                pltpu.make_async_copy(buf, dst, osem).wait()
                pltpu.async_copy(road_hbm.at[idx_vm.at[j + 2]], buf, gsem)
        return carry

    lax.fori_loop(0, QNCH // 2, body, 0)
    pltpu.make_async_copy(
        buf_a, out_hbm.at[pl.ds(obase + (QNCH - 2) * QCH, QCH)], osem_a).wait()
    pltpu.make_async_copy(
        buf_b, out_hbm.at[pl.ds(obase + (QNCH - 1) * QCH, QCH)], osem_b).wait()


_TC1_BLK = 2048


def _tc1_body(nf_ref, w_ref, dg_ref, y_ref):
    x = jnp.dot(nf_ref[...], w_ref[...], preferred_element_type=jnp.float32)
    deg = dg_ref[0, :, 0:1] + dg_ref[1, :, 0:1] + 1.0
    y_ref[...] = x * lax.rsqrt(deg)


def _tc1(node_feat, w, deg2):
    return pl.pallas_call(
        _tc1_body,
        grid=(NPAD // _TC1_BLK,),
        in_specs=[
            pl.BlockSpec((_TC1_BLK, D), lambda i: (i, 0)),
            pl.BlockSpec((D, D), lambda i: (0, 0)),
            pl.BlockSpec((NC, _TC1_BLK, D), lambda i: (0, i, 0)),
        ],
        out_specs=pl.BlockSpec((_TC1_BLK, D), lambda i: (i, 0)),
        out_shape=jax.ShapeDtypeStruct((NPAD, D), jnp.float32),
    )(node_feat, w, deg2)


_TC2_BLK = 512


def _tc2_body(s_ref, dg_ref, y_ref, b_ref, traj_ref, sl_ref, road_ref,
              idxm_ref):
    i = pl.program_id(0)
    deg = dg_ref[0, :, 0:1] + dg_ref[1, :, 0:1] + 1.0
    dinv = lax.rsqrt(deg)
    acc = s_ref[0] + s_ref[1] + y_ref[...]
    val = jnp.maximum(dinv * acc + b_ref[...], 0.0)
    row = i * _TC2_BLK + lax.broadcasted_iota(jnp.int32, (_TC2_BLK, 1), 0)
    road_ref[...] = jnp.where(row < N, val, 0.0)
    pos = lax.broadcasted_iota(jnp.int32, (B, L), 1)
    idxm_ref[...] = jnp.where(pos < sl_ref[...], traj_ref[...], N)


def _tc2(s2, deg2, y, b, traj, seq_len):
    return pl.pallas_call(
        _tc2_body,
        grid=(NPAD // _TC2_BLK,),
        in_specs=[
            pl.BlockSpec((NC, _TC2_BLK, D), lambda i: (0, i, 0)),
            pl.BlockSpec((NC, _TC2_BLK, D), lambda i: (0, i, 0)),
            pl.BlockSpec((_TC2_BLK, D), lambda i: (i, 0)),
            pl.BlockSpec((1, D), lambda i: (0, 0)),
            pl.BlockSpec((B, L), lambda i: (0, 0)),
            pl.BlockSpec((B, 1), lambda i: (0, 0)),
        ],
        out_specs=[
            pl.BlockSpec((_TC2_BLK, D), lambda i: (i, 0)),
            pl.BlockSpec((B, L), lambda i: (0, 0)),
        ],
        out_shape=[
            jax.ShapeDtypeStruct((NPAD, D), jnp.float32),
            jax.ShapeDtypeStruct((B, L), jnp.int32),
        ],
    )(s2, deg2, y, b.reshape(1, D), traj, seq_len.reshape(B, 1))


def _pad_edges(idx):
    pad = jnp.full((EPAD - E,), PADN, dtype=jnp.int32)
    return jnp.concatenate([idx.astype(jnp.int32), pad]).reshape(
        NC, NS, NCHUNK, CH)


def kernel(traj_seqs, seq_len, node_feat, edge_index, W, b):
    src = _pad_edges(edge_index[0])
    dst = _pad_edges(edge_index[1])
    dstf = dst.reshape(NC, ECORE)
    zerosd = jnp.zeros((RPT, D), jnp.float32)

    deg2 = _deg_kernel(dstf)
    y = _tc1(node_feat, W, deg2)
    s2 = _msg_kernel(y, src, dst, zerosd)
    road, idxm = _tc2(s2, deg2, y, b, traj_seqs[..., 0], seq_len)
    out = _traj_kernel(road, idxm.reshape(NW, QNCH, QCH))
    return out.reshape(B, L, D)


# speedup vs baseline: 2.0258x; 2.0258x over previous
import functools

import jax
import jax.numpy as jnp
from jax import lax
from jax.experimental import pallas as pl
from jax.experimental.pallas import tpu as pltpu
from jax.experimental.pallas import tpu_sc as plsc

N = 10000
E = 320000
D = 128
B = 16
L = 2048

NC = 2
NS = 16
NW = NC * NS
CH = 128
NCHUNK = 80
EPW = NCHUNK * CH
EPAD = NW * EPW
NPAD = 10240
PADN = NPAD - 8
RPT = NPAD // NS

TQ = B * L
QP = TQ // NW
QCH = 128
QNCH = QP // QCH

_mesh = functools.partial(
    plsc.VectorSubcoreMesh,
    core_axis_name="c", subcore_axis_name="s", num_cores=NC, num_subcores=NS)


ECORE = EPAD // NC
EPT = ECORE // NS


@functools.partial(
    pl.kernel,
    out_type=jax.ShapeDtypeStruct((NC, NPAD, D), jnp.float32),
    mesh=_mesh(),
    compiler_params=pltpu.CompilerParams(needs_layout_passes=False),
    scratch_types=[
        pltpu.VMEM((EPT,), jnp.int32),
        pltpu.VMEM((NPAD,), jnp.float32),
        pltpu.VMEM((RPT,), jnp.float32),
        pltpu.VMEM((RPT,), jnp.float32),
        pltpu.VMEM((RPT, D), jnp.float32),
        pltpu.VMEM_SHARED((NS, NPAD), jnp.float32),
    ],
)
def _deg_kernel(dstf_hbm, deg_out, dst_vm, acc, rsum, tmp, outbuf, part_sp):
    c = lax.axis_index("c")
    s = lax.axis_index("s")
    pltpu.sync_copy(dstf_hbm.at[c, pl.ds(s * EPT, EPT)], dst_vm)

    zeros16 = jnp.zeros((16,), jnp.float32)

    def zero(i, carry):
        acc[pl.ds(i * 16, 16)] = zeros16
        return carry

    lax.fori_loop(0, NPAD // 16, zero, 0)

    ones16 = jnp.ones((16,), jnp.float32)

    def hist(v, carry):
        idx = dst_vm[pl.ds(v * 16, 16)]
        plsc.addupdate_scatter(acc, [idx], ones16)
        return carry

    lax.fori_loop(0, EPT // 16, hist, 0)

    pltpu.sync_copy(acc, part_sp.at[s])
    plsc.subcore_barrier()

    base = s * RPT

    def red0(i, carry):
        rsum[pl.ds(i * 16, 16)] = zeros16
        return carry

    lax.fori_loop(0, RPT // 16, red0, 0)

    for r in range(NS):
        pltpu.sync_copy(part_sp.at[r, pl.ds(base, RPT)], tmp)

        def radd(i, carry):
            rsum[pl.ds(i * 16, 16)] = (rsum[pl.ds(i * 16, 16)]
                                       + tmp[pl.ds(i * 16, 16)])
            return carry

        lax.fori_loop(0, RPT // 16, radd, 0)

    def bcast(g, carry):
        vec = rsum[pl.ds(g * 16, 16)]
        for lane in range(16):
            splat = jnp.full((16,), vec[lane], jnp.float32)
            for k in range(D // 16):
                outbuf[g * 16 + lane, pl.ds(k * 16, 16)] = splat
        return carry

    lax.fori_loop(0, RPT // 16, bcast, 0)
    pltpu.sync_copy(outbuf, deg_out.at[c, pl.ds(base, RPT)])


@functools.partial(
    pl.kernel,
    out_type=jax.ShapeDtypeStruct((NC, NPAD, D), jnp.float32),
    mesh=_mesh(),
    scratch_types=[
        pltpu.VMEM((NCHUNK // 2, CH), jnp.int32),
        pltpu.VMEM((NCHUNK // 2, CH), jnp.int32),
        pltpu.VMEM((CH, D), jnp.float32),
        pltpu.VMEM((CH, D), jnp.float32),
        pltpu.SemaphoreType.DMA,
        pltpu.SemaphoreType.DMA,
        pltpu.SemaphoreType.DMA,
        pltpu.SemaphoreType.DMA,
        pltpu.VMEM_SHARED((NPAD, D), jnp.float32),
    ],
)
def _msg_kernel(y_hbm, src_hbm, dst_hbm, zeros_hbm, s_out, src_vm, dst_vm,
                buf_a, buf_b, gsem_a, gsem_b, ssem_a, ssem_b, s_sp):
    c = lax.axis_index("c")
    s = lax.axis_index("s")
    base = s * RPT
    pltpu.sync_copy(zeros_hbm, s_sp.at[pl.ds(base, RPT)])
    plsc.subcore_barrier()

    bufs = (buf_a, buf_b)
    gsems = (gsem_a, gsem_b)
    ssems = (ssem_a, ssem_b)
    half = NCHUNK // 2

    for p in range(2):
        pltpu.sync_copy(src_hbm.at[c, s, pl.ds(p * half, half)], src_vm)
        pltpu.sync_copy(dst_hbm.at[c, s, pl.ds(p * half, half)], dst_vm)
        pltpu.async_copy(y_hbm.at[src_vm.at[0]], buf_a, gsem_a)
        pltpu.async_copy(y_hbm.at[src_vm.at[1]], buf_b, gsem_b)

        def body(g, carry):
            for k in range(2):
                j = g * 2 + k
                buf, gsem, ssem = bufs[k], gsems[k], ssems[k]
                pltpu.make_async_copy(y_hbm.at[src_vm.at[j]], buf,
                                      gsem).wait()
                pltpu.async_copy(buf, s_sp.at[dst_vm.at[j]], ssem, add=True)

                @pl.when(j + 2 < half)
                def _():
                    pltpu.make_async_copy(buf, s_sp.at[dst_vm.at[j]],
                                          ssem).wait()
                    pltpu.async_copy(y_hbm.at[src_vm.at[j + 2]], buf, gsem)
            return carry

        lax.fori_loop(0, half // 2, body, 0)
        pltpu.make_async_copy(buf_a, s_sp.at[dst_vm.at[half - 2]],
                              ssem_a).wait()
        pltpu.make_async_copy(buf_b, s_sp.at[dst_vm.at[half - 1]],
                              ssem_b).wait()

    plsc.subcore_barrier()
    pltpu.sync_copy(s_sp.at[pl.ds(base, RPT)], s_out.at[c, pl.ds(base, RPT)])


@functools.partial(
    pl.kernel,
    out_type=jax.ShapeDtypeStruct((TQ, D), jnp.float32),
    mesh=_mesh(),
    scratch_types=[
        pltpu.VMEM((QNCH, QCH), jnp.int32),
        pltpu.VMEM((QCH, D), jnp.float32),
        pltpu.VMEM((QCH, D), jnp.float32),
        pltpu.SemaphoreType.DMA,
        pltpu.SemaphoreType.DMA,
        pltpu.SemaphoreType.DMA,
        pltpu.SemaphoreType.DMA,
        pltpu.VMEM_SHARED((NPAD, D), jnp.float32),
    ],
)
def _traj_kernel(road_hbm, idx_hbm, out_hbm, idx_vm, buf_a, buf_b,
                 gsem_a, gsem_b, osem_a, osem_b, road_sp):
    c = lax.axis_index("c")
    s = lax.axis_index("s")
    wid = s * NC + c
    pltpu.sync_copy(idx_hbm.at[wid], idx_vm)
    base = s * RPT
    pltpu.sync_copy(road_hbm.at[pl.ds(base, RPT)], road_sp.at[pl.ds(base, RPT)])
    plsc.subcore_barrier()
    obase = wid * QP

    bufs = (buf_a, buf_b)
    gsems = (gsem_a, gsem_b)
    osems = (osem_a, osem_b)
    pltpu.async_copy(road_sp.at[idx_vm.at[0]], buf_a, gsem_a)
    pltpu.async_copy(road_sp.at[idx_vm.at[1]], buf_b, gsem_b)

    def body(g, carry):
        for k in range(2):
            j = g * 2 + k
            buf, gsem, osem = bufs[k], gsems[k], osems[k]
            dst = out_hbm.at[pl.ds(obase + j * QCH, QCH)]
            pltpu.make_async_copy(road_sp.at[idx_vm.at[j]], buf, gsem).wait()
            pltpu.async_copy(buf, dst, osem)

            @pl.when(j + 2 < QNCH)
            def _():
                pltpu.make_async_copy(buf, dst, osem).wait()
                pltpu.async_copy(road_sp.at[idx_vm.at[j + 2]], buf, gsem)
        return carry

    lax.fori_loop(0, QNCH // 2, body, 0)
    pltpu.make_async_copy(
        buf_a, out_hbm.at[pl.ds(obase + (QNCH - 2) * QCH, QCH)], osem_a).wait()
    pltpu.make_async_copy(
        buf_b, out_hbm.at[pl.ds(obase + (QNCH - 1) * QCH, QCH)], osem_b).wait()


_TC1_BLK = 2048


def _tc1_body(nf_ref, w_ref, dg_ref, y_ref):
    x = jnp.dot(nf_ref[...], w_ref[...], preferred_element_type=jnp.float32)
    deg = dg_ref[0, :, 0:1] + dg_ref[1, :, 0:1] + 1.0
    y_ref[...] = x * lax.rsqrt(deg)


def _tc1(node_feat, w, deg2):
    return pl.pallas_call(
        _tc1_body,
        grid=(NPAD // _TC1_BLK,),
        in_specs=[
            pl.BlockSpec((_TC1_BLK, D), lambda i: (i, 0)),
            pl.BlockSpec((D, D), lambda i: (0, 0)),
            pl.BlockSpec((NC, _TC1_BLK, D), lambda i: (0, i, 0)),
        ],
        out_specs=pl.BlockSpec((_TC1_BLK, D), lambda i: (i, 0)),
        out_shape=jax.ShapeDtypeStruct((NPAD, D), jnp.float32),
    )(node_feat, w, deg2)


_TC2_BLK = 512


def _tc2_body(s_ref, dg_ref, y_ref, b_ref, traj_ref, sl_ref, road_ref,
              idxm_ref):
    i = pl.program_id(0)
    deg = dg_ref[0, :, 0:1] + dg_ref[1, :, 0:1] + 1.0
    dinv = lax.rsqrt(deg)
    acc = s_ref[0] + s_ref[1] + y_ref[...]
    val = jnp.maximum(dinv * acc + b_ref[...], 0.0)
    row = i * _TC2_BLK + lax.broadcasted_iota(jnp.int32, (_TC2_BLK, 1), 0)
    road_ref[...] = jnp.where(row < N, val, 0.0)
    pos = lax.broadcasted_iota(jnp.int32, (B, L), 1)
    idxm_ref[...] = jnp.where(pos < sl_ref[...], traj_ref[...], N)


def _tc2(s2, deg2, y, b, traj, seq_len):
    return pl.pallas_call(
        _tc2_body,
        grid=(NPAD // _TC2_BLK,),
        in_specs=[
            pl.BlockSpec((NC, _TC2_BLK, D), lambda i: (0, i, 0)),
            pl.BlockSpec((NC, _TC2_BLK, D), lambda i: (0, i, 0)),
            pl.BlockSpec((_TC2_BLK, D), lambda i: (i, 0)),
            pl.BlockSpec((1, D), lambda i: (0, 0)),
            pl.BlockSpec((B, L), lambda i: (0, 0)),
            pl.BlockSpec((B, 1), lambda i: (0, 0)),
        ],
        out_specs=[
            pl.BlockSpec((_TC2_BLK, D), lambda i: (i, 0)),
            pl.BlockSpec((B, L), lambda i: (0, 0)),
        ],
        out_shape=[
            jax.ShapeDtypeStruct((NPAD, D), jnp.float32),
            jax.ShapeDtypeStruct((B, L), jnp.int32),
        ],
    )(s2, deg2, y, b.reshape(1, D), traj, seq_len.reshape(B, 1))


def _pad_edges(idx):
    pad = jnp.full((EPAD - E,), PADN, dtype=jnp.int32)
    return jnp.concatenate([idx.astype(jnp.int32), pad]).reshape(
        NC, NS, NCHUNK, CH)


def kernel(traj_seqs, seq_len, node_feat, edge_index, W, b):
    src = _pad_edges(edge_index[0])
    dst = _pad_edges(edge_index[1])
    dstf = dst.reshape(NC, ECORE)
    zerosd = jnp.zeros((RPT, D), jnp.float32)

    deg2 = _deg_kernel(dstf)
    y = _tc1(node_feat, W, deg2)
    s2 = _msg_kernel(y, src, dst, zerosd)
    road, idxm = _tc2(s2, deg2, y, b, traj_seqs[..., 0], seq_len)
    out = _traj_kernel(road, idxm.reshape(NW, QNCH, QCH))
    return out.reshape(B, L, D)


# speedup vs baseline: 2.4214x; 1.1953x over previous
import functools

import jax
import jax.numpy as jnp
from jax import lax
from jax.experimental import pallas as pl
from jax.experimental.pallas import tpu as pltpu
from jax.experimental.pallas import tpu_sc as plsc

N = 10000
E = 320000
D = 128
B = 16
L = 2048

NC = 2
NS = 16
NW = NC * NS
CH = 128
NCHUNK = 80
EPW = NCHUNK * CH
EPAD = NW * EPW
NPAD = 10240
PADN = NPAD - 8
RPT = NPAD // NS

TQ = B * L
QP = TQ // NW
QCH = 128
QNCH = QP // QCH

_mesh = functools.partial(
    plsc.VectorSubcoreMesh,
    core_axis_name="c", subcore_axis_name="s", num_cores=NC, num_subcores=NS)


ECORE = EPAD // NC
EPT = ECORE // NS


@functools.partial(
    pl.kernel,
    out_type=jax.ShapeDtypeStruct((NC, NPAD, D), jnp.float32),
    mesh=_mesh(),
    compiler_params=pltpu.CompilerParams(needs_layout_passes=False),
    scratch_types=[
        pltpu.VMEM((EPT,), jnp.int32),
        pltpu.VMEM((NPAD,), jnp.float32),
        pltpu.VMEM((RPT,), jnp.float32),
        pltpu.VMEM((RPT,), jnp.float32),
        pltpu.VMEM((RPT, D), jnp.float32),
        pltpu.VMEM_SHARED((NS, NPAD), jnp.float32),
    ],
)
def _deg_kernel(dstf_hbm, deg_out, dst_vm, acc, rsum, tmp, outbuf, part_sp):
    c = lax.axis_index("c")
    s = lax.axis_index("s")
    pltpu.sync_copy(dstf_hbm.at[c, pl.ds(s * EPT, EPT)], dst_vm)

    zeros16 = jnp.zeros((16,), jnp.float32)

    def zero(i, carry):
        acc[pl.ds(i * 16, 16)] = zeros16
        return carry

    lax.fori_loop(0, NPAD // 16, zero, 0)

    ones16 = jnp.ones((16,), jnp.float32)

    def hist(v, carry):
        idx = dst_vm[pl.ds(v * 16, 16)]
        plsc.addupdate_scatter(acc, [idx], ones16)
        return carry

    lax.fori_loop(0, EPT // 16, hist, 0)

    pltpu.sync_copy(acc, part_sp.at[s])
    plsc.subcore_barrier()

    base = s * RPT

    def red0(i, carry):
        rsum[pl.ds(i * 16, 16)] = zeros16
        return carry

    lax.fori_loop(0, RPT // 16, red0, 0)

    for r in range(NS):
        pltpu.sync_copy(part_sp.at[r, pl.ds(base, RPT)], tmp)

        def radd(i, carry):
            rsum[pl.ds(i * 16, 16)] = (rsum[pl.ds(i * 16, 16)]
                                       + tmp[pl.ds(i * 16, 16)])
            return carry

        lax.fori_loop(0, RPT // 16, radd, 0)

    def bcast(g, carry):
        vec = rsum[pl.ds(g * 16, 16)]
        for lane in range(16):
            splat = jnp.full((16,), vec[lane], jnp.float32)
            for k in range(D // 16):
                outbuf[g * 16 + lane, pl.ds(k * 16, 16)] = splat
        return carry

    lax.fori_loop(0, RPT // 16, bcast, 0)
    pltpu.sync_copy(outbuf, deg_out.at[c, pl.ds(base, RPT)])


ECH = 1024
ENCH = 160
PAIRS = 4


@functools.partial(
    pl.kernel,
    out_type=jax.ShapeDtypeStruct((NC, D * NPAD), jnp.float32),
    mesh=_mesh(),
    compiler_params=pltpu.CompilerParams(needs_layout_passes=False),
    scratch_types=[
        pltpu.VMEM((PAIRS * NPAD,), jnp.int32),
        pltpu.VMEM((2 * PAIRS * NPAD,), jnp.float32),
        pltpu.VMEM((ECH,), jnp.int32),
        pltpu.VMEM((ECH,), jnp.int32),
        pltpu.VMEM((ECH,), jnp.int32),
        pltpu.VMEM((ECH,), jnp.int32),
        pltpu.SemaphoreType.DMA,
        pltpu.SemaphoreType.DMA,
    ],
)
def _msg_kernel(ypack_hbm, srcf_hbm, dstf_hbm, st_out, ystrip, acc,
                src_a, dst_a, src_b, dst_b, sem_a, sem_b):
    c = lax.axis_index("c")
    s = lax.axis_index("s")
    pltpu.sync_copy(ypack_hbm.at[pl.ds(s * PAIRS * NPAD, PAIRS * NPAD)],
                    ystrip)

    zeros16 = jnp.zeros((16,), jnp.float32)

    def zero(i, carry):
        acc[pl.ds(i * 16, 16)] = zeros16
        return carry

    lax.fori_loop(0, 2 * PAIRS * NPAD // 16, zero, 0)

    srcs = (src_a, src_b)
    dsts = (dst_a, dst_b)
    sems = (sem_a, sem_b)
    hi_mask = jnp.int32(-65536)

    pltpu.async_copy(srcf_hbm.at[c, pl.ds(0, ECH)], src_a, sem_a)
    pltpu.async_copy(dstf_hbm.at[c, pl.ds(0, ECH)], dst_a, sem_a)
    pltpu.async_copy(srcf_hbm.at[c, pl.ds(ECH, ECH)], src_b, sem_b)
    pltpu.async_copy(dstf_hbm.at[c, pl.ds(ECH, ECH)], dst_b, sem_b)

    def chunk_body(g, carry):
        for k in range(2):
            ch = g * 2 + k
            src_vm, dst_vm, sem = srcs[k], dsts[k], sems[k]
            off = ch * ECH
            pltpu.make_async_copy(srcf_hbm.at[c, pl.ds(off, ECH)], src_vm,
                                  sem).wait()
            pltpu.make_async_copy(dstf_hbm.at[c, pl.ds(off, ECH)], dst_vm,
                                  sem).wait()

            def edge_body(v, carry2):
                sv = src_vm[pl.ds(v * 16, 16)]
                dv = dst_vm[pl.ds(v * 16, 16)]
                for pr in range(PAIRS):
                    pairs = plsc.load_gather(ystrip, [sv + pr * NPAD])
                    ev = plsc.bitcast(jnp.left_shift(pairs, 16), jnp.float32)
                    ov = plsc.bitcast(jnp.bitwise_and(pairs, hi_mask),
                                      jnp.float32)
                    plsc.addupdate_scatter(acc, [dv + (2 * pr) * NPAD], ev)
                    plsc.addupdate_scatter(acc, [dv + (2 * pr + 1) * NPAD],
                                           ov)
                return carry2

            lax.fori_loop(0, ECH // 16, edge_body, 0)

            @pl.when(ch + 2 < ENCH)
            def _():
                noff = (ch + 2) * ECH
                pltpu.async_copy(srcf_hbm.at[c, pl.ds(noff, ECH)], src_vm,
                                 sem)
                pltpu.async_copy(dstf_hbm.at[c, pl.ds(noff, ECH)], dst_vm,
                                 sem)
        return carry

    lax.fori_loop(0, ENCH // 2, chunk_body, 0)
    pltpu.sync_copy(acc,
                    st_out.at[c, pl.ds(s * 2 * PAIRS * NPAD,
                                       2 * PAIRS * NPAD)])


@functools.partial(
    pl.kernel,
    out_type=jax.ShapeDtypeStruct((TQ, D), jnp.float32),
    mesh=_mesh(),
    scratch_types=[
        pltpu.VMEM((QNCH, QCH), jnp.int32),
        pltpu.VMEM((QCH, D), jnp.float32),
        pltpu.VMEM((QCH, D), jnp.float32),
        pltpu.SemaphoreType.DMA,
        pltpu.SemaphoreType.DMA,
        pltpu.SemaphoreType.DMA,
        pltpu.SemaphoreType.DMA,
        pltpu.VMEM_SHARED((NPAD, D), jnp.float32),
    ],
)
def _traj_kernel(road_hbm, idx_hbm, out_hbm, idx_vm, buf_a, buf_b,
                 gsem_a, gsem_b, osem_a, osem_b, road_sp):
    c = lax.axis_index("c")
    s = lax.axis_index("s")
    wid = s * NC + c
    pltpu.sync_copy(idx_hbm.at[wid], idx_vm)
    base = s * RPT
    pltpu.sync_copy(road_hbm.at[pl.ds(base, RPT)], road_sp.at[pl.ds(base, RPT)])
    plsc.subcore_barrier()
    obase = wid * QP

    bufs = (buf_a, buf_b)
    gsems = (gsem_a, gsem_b)
    osems = (osem_a, osem_b)
    pltpu.async_copy(road_sp.at[idx_vm.at[0]], buf_a, gsem_a)
    pltpu.async_copy(road_sp.at[idx_vm.at[1]], buf_b, gsem_b)

    def body(g, carry):
        for k in range(2):
            j = g * 2 + k
            buf, gsem, osem = bufs[k], gsems[k], osems[k]
            dst = out_hbm.at[pl.ds(obase + j * QCH, QCH)]
            pltpu.make_async_copy(road_sp.at[idx_vm.at[j]], buf, gsem).wait()
            pltpu.async_copy(buf, dst, osem)

            @pl.when(j + 2 < QNCH)
            def _():
                pltpu.make_async_copy(buf, dst, osem).wait()
                pltpu.async_copy(road_sp.at[idx_vm.at[j + 2]], buf, gsem)
        return carry

    lax.fori_loop(0, QNCH // 2, body, 0)
    pltpu.make_async_copy(
        buf_a, out_hbm.at[pl.ds(obase + (QNCH - 2) * QCH, QCH)], osem_a).wait()
    pltpu.make_async_copy(
        buf_b, out_hbm.at[pl.ds(obase + (QNCH - 1) * QCH, QCH)], osem_b).wait()


_TC1_BLK = 2048


def _tc1_body(nf_ref, w_ref, dg_ref, y_ref, yp_ref):
    x = jnp.dot(nf_ref[...], w_ref[...], preferred_element_type=jnp.float32)
    deg = dg_ref[0, :, 0:1] + dg_ref[1, :, 0:1] + 1.0
    y = x * lax.rsqrt(deg)
    y_ref[...] = y
    rows = lax.broadcasted_iota(jnp.int32, (D // 2, D), 0)
    cols = lax.broadcasted_iota(jnp.int32, (D // 2, D), 1)
    q_even = (cols == 2 * rows).astype(jnp.float32)
    q_odd = (cols == 2 * rows + 1).astype(jnp.float32)
    dn = (((1,), (1,)), ((), ()))
    ye = lax.dot_general(q_even, y, dn, preferred_element_type=jnp.float32)
    yo = lax.dot_general(q_odd, y, dn, preferred_element_type=jnp.float32)
    be = lax.shift_right_logical(lax.bitcast_convert_type(ye, jnp.int32), 16)
    bo = jnp.bitwise_and(lax.bitcast_convert_type(yo, jnp.int32),
                         jnp.int32(-65536))
    yp_ref[...] = jnp.bitwise_or(be, bo)


def _tc1(node_feat, w, deg2):
    return pl.pallas_call(
        _tc1_body,
        grid=(NPAD // _TC1_BLK,),
        in_specs=[
            pl.BlockSpec((_TC1_BLK, D), lambda i: (i, 0)),
            pl.BlockSpec((D, D), lambda i: (0, 0)),
            pl.BlockSpec((NC, _TC1_BLK, D), lambda i: (0, i, 0)),
        ],
        out_specs=[
            pl.BlockSpec((_TC1_BLK, D), lambda i: (i, 0)),
            pl.BlockSpec((D // 2, _TC1_BLK), lambda i: (0, i)),
        ],
        out_shape=[
            jax.ShapeDtypeStruct((NPAD, D), jnp.float32),
            jax.ShapeDtypeStruct((D // 2, NPAD), jnp.int32),
        ],
    )(node_feat, w, deg2)


_TC2_BLK = 512


def _tc2_body(s_ref, dg_ref, y_ref, b_ref, traj_ref, sl_ref, road_ref,
              idxm_ref):
    i = pl.program_id(0)
    deg = dg_ref[0, :, 0:1] + dg_ref[1, :, 0:1] + 1.0
    dinv = lax.rsqrt(deg)
    st = s_ref[0] + s_ref[1]
    rows = lax.broadcasted_iota(jnp.int32, (D, D), 0)
    cols = lax.broadcasted_iota(jnp.int32, (D, D), 1)
    eye = (rows == cols).astype(jnp.float32)
    dn = (((0,), (0,)), ((), ()))
    s_blk = lax.dot_general(st, eye, dn,
                            preferred_element_type=jnp.float32)
    acc = s_blk + y_ref[...]
    val = jnp.maximum(dinv * acc + b_ref[...], 0.0)
    row = i * _TC2_BLK + lax.broadcasted_iota(jnp.int32, (_TC2_BLK, 1), 0)
    road_ref[...] = jnp.where(row < N, val, 0.0)
    pos = lax.broadcasted_iota(jnp.int32, (B, L), 1)
    idxm_ref[...] = jnp.where(pos < sl_ref[...], traj_ref[...], N)


def _tc2(s2, deg2, y, b, traj, seq_len):
    return pl.pallas_call(
        _tc2_body,
        grid=(NPAD // _TC2_BLK,),
        in_specs=[
            pl.BlockSpec((NC, D, _TC2_BLK), lambda i: (0, 0, i)),
            pl.BlockSpec((NC, _TC2_BLK, D), lambda i: (0, i, 0)),
            pl.BlockSpec((_TC2_BLK, D), lambda i: (i, 0)),
            pl.BlockSpec((1, D), lambda i: (0, 0)),
            pl.BlockSpec((B, L), lambda i: (0, 0)),
            pl.BlockSpec((B, 1), lambda i: (0, 0)),
        ],
        out_specs=[
            pl.BlockSpec((_TC2_BLK, D), lambda i: (i, 0)),
            pl.BlockSpec((B, L), lambda i: (0, 0)),
        ],
        out_shape=[
            jax.ShapeDtypeStruct((NPAD, D), jnp.float32),
            jax.ShapeDtypeStruct((B, L), jnp.int32),
        ],
    )(s2, deg2, y, b.reshape(1, D), traj, seq_len.reshape(B, 1))


def _pad_edges(idx):
    pad = jnp.full((EPAD - E,), PADN, dtype=jnp.int32)
    return jnp.concatenate([idx.astype(jnp.int32), pad]).reshape(NC, ECORE)


def kernel(traj_seqs, seq_len, node_feat, edge_index, W, b):
    srcf = _pad_edges(edge_index[0])
    dstf = _pad_edges(edge_index[1])

    deg2 = _deg_kernel(dstf)
    y, ypack = _tc1(node_feat, W, deg2)
    st = _msg_kernel(ypack.reshape(-1), srcf, dstf)
    s2 = st.reshape(NC, D, NPAD)
    road, idxm = _tc2(s2, deg2, y, b, traj_seqs[..., 0], seq_len)
    out = _traj_kernel(road, idxm.reshape(NW, QNCH, QCH))
    return out.reshape(B, L, D)


# speedup vs baseline: 2.4358x; 1.0060x over previous
import functools

import jax
import jax.numpy as jnp
from jax import lax
from jax.experimental import pallas as pl
from jax.experimental.pallas import tpu as pltpu
from jax.experimental.pallas import tpu_sc as plsc

N = 10000
E = 320000
D = 128
B = 16
L = 2048

NC = 2
NS = 16
NW = NC * NS
CH = 128
NCHUNK = 80
EPW = NCHUNK * CH
EPAD = NW * EPW
NPAD = 10240
PADN = NPAD - 8
RPT = NPAD // NS

TQ = B * L
QP = TQ // NW
QCH = 128
QNCH = QP // QCH

_mesh = functools.partial(
    plsc.VectorSubcoreMesh,
    core_axis_name="c", subcore_axis_name="s", num_cores=NC, num_subcores=NS)


ECORE = EPAD // NC
EPT = ECORE // NS


@functools.partial(
    pl.kernel,
    out_type=jax.ShapeDtypeStruct((NC, NPAD, D), jnp.float32),
    mesh=_mesh(),
    compiler_params=pltpu.CompilerParams(needs_layout_passes=False),
    scratch_types=[
        pltpu.VMEM((EPT,), jnp.int32),
        pltpu.VMEM((NPAD,), jnp.float32),
        pltpu.VMEM((RPT,), jnp.float32),
        pltpu.VMEM((RPT,), jnp.float32),
        pltpu.VMEM((RPT, D), jnp.float32),
        pltpu.VMEM_SHARED((NS, NPAD), jnp.float32),
    ],
)
def _deg_kernel(dstf_hbm, deg_out, dst_vm, acc, rsum, tmp, outbuf, part_sp):
    c = lax.axis_index("c")
    s = lax.axis_index("s")
    pltpu.sync_copy(dstf_hbm.at[c, pl.ds(s * EPT, EPT)], dst_vm)

    zeros16 = jnp.zeros((16,), jnp.float32)

    def zero(i, carry):
        acc[pl.ds(i * 16, 16)] = zeros16
        return carry

    lax.fori_loop(0, NPAD // 16, zero, 0)

    ones16 = jnp.ones((16,), jnp.float32)

    def hist(v, carry):
        idx = dst_vm[pl.ds(v * 16, 16)]
        plsc.addupdate_scatter(acc, [idx], ones16)
        return carry

    lax.fori_loop(0, EPT // 16, hist, 0)

    pltpu.sync_copy(acc, part_sp.at[s])
    plsc.subcore_barrier()

    base = s * RPT

    def red0(i, carry):
        rsum[pl.ds(i * 16, 16)] = zeros16
        return carry

    lax.fori_loop(0, RPT // 16, red0, 0)

    for r in range(NS):
        pltpu.sync_copy(part_sp.at[r, pl.ds(base, RPT)], tmp)

        def radd(i, carry):
            rsum[pl.ds(i * 16, 16)] = (rsum[pl.ds(i * 16, 16)]
                                       + tmp[pl.ds(i * 16, 16)])
            return carry

        lax.fori_loop(0, RPT // 16, radd, 0)

    def bcast(g, carry):
        vec = rsum[pl.ds(g * 16, 16)]
        for lane in range(16):
            splat = jnp.full((16,), vec[lane], jnp.float32)
            for k in range(D // 16):
                outbuf[g * 16 + lane, pl.ds(k * 16, 16)] = splat
        return carry

    lax.fori_loop(0, RPT // 16, bcast, 0)
    pltpu.sync_copy(outbuf, deg_out.at[c, pl.ds(base, RPT)])


ECH = 1024
ENCH = 160
PAIRS = 4


@functools.partial(
    pl.kernel,
    out_type=jax.ShapeDtypeStruct((NC, D * NPAD), jnp.float32),
    mesh=_mesh(),
    compiler_params=pltpu.CompilerParams(needs_layout_passes=False),
    scratch_types=[
        pltpu.VMEM((PAIRS * NPAD,), jnp.int32),
        pltpu.VMEM((2 * PAIRS * NPAD,), jnp.float32),
        pltpu.VMEM((ECH,), jnp.int32),
        pltpu.VMEM((ECH,), jnp.int32),
        pltpu.VMEM((ECH,), jnp.int32),
        pltpu.VMEM((ECH,), jnp.int32),
        pltpu.SemaphoreType.DMA,
        pltpu.SemaphoreType.DMA,
    ],
)
def _msg_kernel(ypack_hbm, srcf_hbm, dstf_hbm, st_out, ystrip, acc,
                src_a, dst_a, src_b, dst_b, sem_a, sem_b):
    c = lax.axis_index("c")
    s = lax.axis_index("s")
    pltpu.sync_copy(ypack_hbm.at[pl.ds(s * PAIRS * NPAD, PAIRS * NPAD)],
                    ystrip)

    zeros16 = jnp.zeros((16,), jnp.float32)

    def zero(i, carry):
        acc[pl.ds(i * 16, 16)] = zeros16
        return carry

    lax.fori_loop(0, 2 * PAIRS * NPAD // 16, zero, 0)

    srcs = (src_a, src_b)
    dsts = (dst_a, dst_b)
    sems = (sem_a, sem_b)
    hi_mask = jnp.int32(-65536)

    pltpu.async_copy(srcf_hbm.at[c, pl.ds(0, ECH)], src_a, sem_a)
    pltpu.async_copy(dstf_hbm.at[c, pl.ds(0, ECH)], dst_a, sem_a)
    pltpu.async_copy(srcf_hbm.at[c, pl.ds(ECH, ECH)], src_b, sem_b)
    pltpu.async_copy(dstf_hbm.at[c, pl.ds(ECH, ECH)], dst_b, sem_b)

    def chunk_body(g, carry):
        for k in range(2):
            ch = g * 2 + k
            src_vm, dst_vm, sem = srcs[k], dsts[k], sems[k]
            off = ch * ECH
            pltpu.make_async_copy(srcf_hbm.at[c, pl.ds(off, ECH)], src_vm,
                                  sem).wait()
            pltpu.make_async_copy(dstf_hbm.at[c, pl.ds(off, ECH)], dst_vm,
                                  sem).wait()

            def edge_body(g2, carry2):
                for u in range(4):
                    v = g2 * 4 + u
                    sv = src_vm[pl.ds(v * 16, 16)]
                    dv = dst_vm[pl.ds(v * 16, 16)]
                    for pr in range(PAIRS):
                        pairs = plsc.load_gather(ystrip, [sv + pr * NPAD])
                        ev = plsc.bitcast(jnp.left_shift(pairs, 16),
                                          jnp.float32)
                        ov = plsc.bitcast(jnp.bitwise_and(pairs, hi_mask),
                                          jnp.float32)
                        plsc.addupdate_scatter(acc, [dv + (2 * pr) * NPAD],
                                               ev)
                        plsc.addupdate_scatter(acc,
                                               [dv + (2 * pr + 1) * NPAD],
                                               ov)
                return carry2

            lax.fori_loop(0, ECH // 64, edge_body, 0)

            @pl.when(ch + 2 < ENCH)
            def _():
                noff = (ch + 2) * ECH
                pltpu.async_copy(srcf_hbm.at[c, pl.ds(noff, ECH)], src_vm,
                                 sem)
                pltpu.async_copy(dstf_hbm.at[c, pl.ds(noff, ECH)], dst_vm,
                                 sem)
        return carry

    lax.fori_loop(0, ENCH // 2, chunk_body, 0)
    pltpu.sync_copy(acc,
                    st_out.at[c, pl.ds(s * 2 * PAIRS * NPAD,
                                       2 * PAIRS * NPAD)])


@functools.partial(
    pl.kernel,
    out_type=jax.ShapeDtypeStruct((TQ, D), jnp.float32),
    mesh=_mesh(),
    scratch_types=[
        pltpu.VMEM((QNCH, QCH), jnp.int32),
        pltpu.VMEM((QCH, D), jnp.float32),
        pltpu.VMEM((QCH, D), jnp.float32),
        pltpu.SemaphoreType.DMA,
        pltpu.SemaphoreType.DMA,
        pltpu.SemaphoreType.DMA,
        pltpu.SemaphoreType.DMA,
        pltpu.VMEM_SHARED((NPAD, D), jnp.float32),
    ],
)
def _traj_kernel(road_hbm, idx_hbm, out_hbm, idx_vm, buf_a, buf_b,
                 gsem_a, gsem_b, osem_a, osem_b, road_sp):
    c = lax.axis_index("c")
    s = lax.axis_index("s")
    wid = s * NC + c
    pltpu.sync_copy(idx_hbm.at[wid], idx_vm)
    base = s * RPT
    pltpu.sync_copy(road_hbm.at[pl.ds(base, RPT)], road_sp.at[pl.ds(base, RPT)])
    plsc.subcore_barrier()
    obase = wid * QP

    bufs = (buf_a, buf_b)
    gsems = (gsem_a, gsem_b)
    osems = (osem_a, osem_b)
    pltpu.async_copy(road_sp.at[idx_vm.at[0]], buf_a, gsem_a)
    pltpu.async_copy(road_sp.at[idx_vm.at[1]], buf_b, gsem_b)

    def body(g, carry):
        for k in range(2):
            j = g * 2 + k
            buf, gsem, osem = bufs[k], gsems[k], osems[k]
            dst = out_hbm.at[pl.ds(obase + j * QCH, QCH)]
            pltpu.make_async_copy(road_sp.at[idx_vm.at[j]], buf, gsem).wait()
            pltpu.async_copy(buf, dst, osem)

            @pl.when(j + 2 < QNCH)
            def _():
                pltpu.make_async_copy(buf, dst, osem).wait()
                pltpu.async_copy(road_sp.at[idx_vm.at[j + 2]], buf, gsem)
        return carry

    lax.fori_loop(0, QNCH // 2, body, 0)
    pltpu.make_async_copy(
        buf_a, out_hbm.at[pl.ds(obase + (QNCH - 2) * QCH, QCH)], osem_a).wait()
    pltpu.make_async_copy(
        buf_b, out_hbm.at[pl.ds(obase + (QNCH - 1) * QCH, QCH)], osem_b).wait()


_TC1_BLK = 2048


def _tc1_body(nf_ref, w_ref, dg_ref, y_ref, yp_ref):
    x = jnp.dot(nf_ref[...], w_ref[...], preferred_element_type=jnp.float32)
    deg = dg_ref[0, :, 0:1] + dg_ref[1, :, 0:1] + 1.0
    y = x * lax.rsqrt(deg)
    y_ref[...] = y
    rows = lax.broadcasted_iota(jnp.int32, (D // 2, D), 0)
    cols = lax.broadcasted_iota(jnp.int32, (D // 2, D), 1)
    q_even = (cols == 2 * rows).astype(jnp.float32)
    q_odd = (cols == 2 * rows + 1).astype(jnp.float32)
    dn = (((1,), (1,)), ((), ()))
    ye = lax.dot_general(q_even, y, dn, preferred_element_type=jnp.float32)
    yo = lax.dot_general(q_odd, y, dn, preferred_element_type=jnp.float32)
    be = lax.shift_right_logical(lax.bitcast_convert_type(ye, jnp.int32), 16)
    bo = jnp.bitwise_and(lax.bitcast_convert_type(yo, jnp.int32),
                         jnp.int32(-65536))
    yp_ref[...] = jnp.bitwise_or(be, bo)


def _tc1(node_feat, w, deg2):
    return pl.pallas_call(
        _tc1_body,
        grid=(NPAD // _TC1_BLK,),
        in_specs=[
            pl.BlockSpec((_TC1_BLK, D), lambda i: (i, 0)),
            pl.BlockSpec((D, D), lambda i: (0, 0)),
            pl.BlockSpec((NC, _TC1_BLK, D), lambda i: (0, i, 0)),
        ],
        out_specs=[
            pl.BlockSpec((_TC1_BLK, D), lambda i: (i, 0)),
            pl.BlockSpec((D // 2, _TC1_BLK), lambda i: (0, i)),
        ],
        out_shape=[
            jax.ShapeDtypeStruct((NPAD, D), jnp.float32),
            jax.ShapeDtypeStruct((D // 2, NPAD), jnp.int32),
        ],
    )(node_feat, w, deg2)


_TC2_BLK = 512


def _tc2_body(s_ref, dg_ref, y_ref, b_ref, traj_ref, sl_ref, road_ref,
              idxm_ref):
    i = pl.program_id(0)
    deg = dg_ref[0, :, 0:1] + dg_ref[1, :, 0:1] + 1.0
    dinv = lax.rsqrt(deg)
    st = s_ref[0] + s_ref[1]
    rows = lax.broadcasted_iota(jnp.int32, (D, D), 0)
    cols = lax.broadcasted_iota(jnp.int32, (D, D), 1)
    eye = (rows == cols).astype(jnp.float32)
    dn = (((0,), (0,)), ((), ()))
    s_blk = lax.dot_general(st, eye, dn,
                            preferred_element_type=jnp.float32)
    acc = s_blk + y_ref[...]
    val = jnp.maximum(dinv * acc + b_ref[...], 0.0)
    row = i * _TC2_BLK + lax.broadcasted_iota(jnp.int32, (_TC2_BLK, 1), 0)
    road_ref[...] = jnp.where(row < N, val, 0.0)
    pos = lax.broadcasted_iota(jnp.int32, (B, L), 1)
    idxm_ref[...] = jnp.where(pos < sl_ref[...], traj_ref[...], N)


def _tc2(s2, deg2, y, b, traj, seq_len):
    return pl.pallas_call(
        _tc2_body,
        grid=(NPAD // _TC2_BLK,),
        in_specs=[
            pl.BlockSpec((NC, D, _TC2_BLK), lambda i: (0, 0, i)),
            pl.BlockSpec((NC, _TC2_BLK, D), lambda i: (0, i, 0)),
            pl.BlockSpec((_TC2_BLK, D), lambda i: (i, 0)),
            pl.BlockSpec((1, D), lambda i: (0, 0)),
            pl.BlockSpec((B, L), lambda i: (0, 0)),
            pl.BlockSpec((B, 1), lambda i: (0, 0)),
        ],
        out_specs=[
            pl.BlockSpec((_TC2_BLK, D), lambda i: (i, 0)),
            pl.BlockSpec((B, L), lambda i: (0, 0)),
        ],
        out_shape=[
            jax.ShapeDtypeStruct((NPAD, D), jnp.float32),
            jax.ShapeDtypeStruct((B, L), jnp.int32),
        ],
    )(s2, deg2, y, b.reshape(1, D), traj, seq_len.reshape(B, 1))


def _pad_edges(idx):
    pad = jnp.full((EPAD - E,), PADN, dtype=jnp.int32)
    return jnp.concatenate([idx.astype(jnp.int32), pad]).reshape(NC, ECORE)


def kernel(traj_seqs, seq_len, node_feat, edge_index, W, b):
    srcf = _pad_edges(edge_index[0])
    dstf = _pad_edges(edge_index[1])

    deg2 = _deg_kernel(dstf)
    y, ypack = _tc1(node_feat, W, deg2)
    st = _msg_kernel(ypack.reshape(-1), srcf, dstf)
    s2 = st.reshape(NC, D, NPAD)
    road, idxm = _tc2(s2, deg2, y, b, traj_seqs[..., 0], seq_len)
    out = _traj_kernel(road, idxm.reshape(NW, QNCH, QCH))
    return out.reshape(B, L, D)
